# Initial kernel scaffold; baseline (speedup 1.0000x reference)
import jax
import jax.numpy as jnp
from jax.experimental import pallas as pl


def kernel(features, edge_index, W1, b1, W2, b2, W3, b3, gamma, beta, mW1, mb1, mW2, mb2):
    raise NotImplementedError("write your pallas kernel here")


# speedup vs baseline: 3.4284x; 3.4284x over previous
import functools

import jax
import jax.numpy as jnp
from jax import lax
from jax.experimental import pallas as pl
from jax.experimental.pallas import tpu as pltpu
from jax.experimental.pallas import tpu_sc as plsc

N = 10000
D = 128
E = 320000
NC = 2
NS = 16
NW = NC * NS
CHUNK = 128
CPW = 79
EPW = CPW * CHUNK
EP = NW * EPW
NACC = 10240
ROWS_PT = NACC // NS
DUMMY = N + 8

_mesh = plsc.VectorSubcoreMesh(core_axis_name="c", subcore_axis_name="s")


def _wid():
    return lax.axis_index("s") * NC + lax.axis_index("c")


@functools.partial(
    pl.kernel,
    out_type=jax.ShapeDtypeStruct((NC, NACC, D), jnp.float32),
    mesh=_mesh,
    scratch_types=[
        pltpu.VMEM((CHUNK,), jnp.int32),
        pltpu.VMEM((CHUNK, D), jnp.float32),
        pltpu.VMEM_SHARED((NACC, D), jnp.float32),
    ],
)
def _deg_kernel(dst_hbm, zeros_hbm, ones_hbm, out_hbm, idx_d, ones_v, acc_sh):
    c = lax.axis_index("c")
    s = lax.axis_index("s")
    wid = _wid()
    pltpu.sync_copy(zeros_hbm, acc_sh.at[pl.ds(s * ROWS_PT, ROWS_PT)])
    pltpu.sync_copy(ones_hbm, ones_v)
    plsc.subcore_barrier()
    base = wid * EPW

    def body(j, carry):
        off = pl.multiple_of(base + j * CHUNK, CHUNK)
        pltpu.sync_copy(dst_hbm.at[pl.ds(off, CHUNK)], idx_d)
        pltpu.sync_copy(ones_v, acc_sh.at[idx_d], add=True)
        return carry

    lax.fori_loop(0, CPW, body, 0)
    plsc.subcore_barrier()
    pltpu.sync_copy(
        acc_sh.at[pl.ds(s * ROWS_PT, ROWS_PT)],
        out_hbm.at[c, pl.ds(s * ROWS_PT, ROWS_PT)],
    )


@functools.partial(
    pl.kernel,
    out_type=jax.ShapeDtypeStruct((NC, NACC, D), jnp.float32),
    mesh=_mesh,
    scratch_types=[
        pltpu.VMEM((CHUNK,), jnp.int32),
        pltpu.VMEM((CHUNK,), jnp.int32),
        pltpu.VMEM((CHUNK, D), jnp.float32),
        pltpu.VMEM_SHARED((NACC, D), jnp.float32),
        pltpu.SemaphoreType.DMA,
    ],
)
def _spmm_kernel(g_hbm, src_hbm, dst_hbm, zeros_hbm, out_hbm,
                 idx_s, idx_d, rows_v, acc_sh, sem):
    c = lax.axis_index("c")
    s = lax.axis_index("s")
    wid = _wid()
    pltpu.sync_copy(zeros_hbm, acc_sh.at[pl.ds(s * ROWS_PT, ROWS_PT)])
    plsc.subcore_barrier()
    base = wid * EPW

    def body(j, carry):
        off = pl.multiple_of(base + j * CHUNK, CHUNK)
        pltpu.sync_copy(src_hbm.at[pl.ds(off, CHUNK)], idx_s)
        pltpu.sync_copy(dst_hbm.at[pl.ds(off, CHUNK)], idx_d)
        pltpu.async_copy(g_hbm.at[idx_s], rows_v, sem).wait()
        pltpu.sync_copy(rows_v, acc_sh.at[idx_d], add=True)
        return carry

    lax.fori_loop(0, CPW, body, 0)
    plsc.subcore_barrier()
    pltpu.sync_copy(
        acc_sh.at[pl.ds(s * ROWS_PT, ROWS_PT)],
        out_hbm.at[c, pl.ds(s * ROWS_PT, ROWS_PT)],
    )


def _tc1_body(dega_ref, degb_ref, feat_ref, norm_ref, g1_ref):
    deg = dega_ref[...][:, 0:1] + degb_ref[...][:, 0:1]
    norm = lax.rsqrt(jnp.maximum(deg, 1.0))
    normb = jnp.broadcast_to(norm, (N, D))
    norm_ref[...] = normb
    g1_ref[...] = feat_ref[...] * normb


def _tc2_body(a1a_ref, a1b_ref, norm_ref, tx1_ref, g2_ref):
    norm = norm_ref[...]
    tx1 = -norm * (a1a_ref[...] + a1b_ref[...])
    tx1_ref[...] = tx1
    g2_ref[...] = norm * tx1


def _tc3_body(x_ref, tx1_ref, a2a_ref, a2b_ref, norm_ref, w1_ref, b1_ref,
              gamma_ref, beta_ref, hbn_ref, g3_ref):
    x = x_ref[...]
    norm = norm_ref[...]
    tx1 = tx1_ref[...]
    tx2 = -2.0 * norm * (a2a_ref[...] + a2b_ref[...]) - x
    pre = (jnp.dot(x, w1_ref[0], preferred_element_type=jnp.float32)
           + jnp.dot(tx1, w1_ref[1], preferred_element_type=jnp.float32)
           + jnp.dot(tx2, w1_ref[2], preferred_element_type=jnp.float32)
           + b1_ref[...])
    h = jnp.maximum(pre, 0.0)
    mean = jnp.mean(h, axis=0, keepdims=True)
    var = jnp.mean((h - mean) ** 2, axis=0, keepdims=True)
    hbn = (h - mean) * lax.rsqrt(var + 1e-5) * gamma_ref[...] + beta_ref[...]
    hbn_ref[...] = hbn
    g3_ref[...] = norm * hbn


def _tc4_body(a3a_ref, a3b_ref, norm_ref, t1_ref, g4_ref):
    norm = norm_ref[...]
    t1 = -norm * (a3a_ref[...] + a3b_ref[...])
    t1_ref[...] = t1
    g4_ref[...] = norm * t1


def _tc5_body(hbn_ref, t1_ref, a4a_ref, a4b_ref, norm_ref, w3_ref, b3_ref,
              mw1_ref, mb1_ref, mw2_ref, mb2_ref, out_ref):
    hbn = hbn_ref[...]
    norm = norm_ref[...]
    t1 = t1_ref[...]
    t2 = -2.0 * norm * (a4a_ref[...] + a4b_ref[...]) - hbn
    h3 = (jnp.dot(hbn, w3_ref[0], preferred_element_type=jnp.float32)
          + jnp.dot(t1, w3_ref[1], preferred_element_type=jnp.float32)
          + jnp.dot(t2, w3_ref[2], preferred_element_type=jnp.float32)
          + b3_ref[...])
    h4 = jnp.maximum(h3, 0.0) + hbn
    hid = jnp.maximum(
        jnp.dot(h4, mw1_ref[...], preferred_element_type=jnp.float32)
        + mb1_ref[...], 0.0)
    out_ref[...] = (jnp.dot(hid, mw2_ref[...], preferred_element_type=jnp.float32)
                    + mb2_ref[...])


def _tc_call(body, n_out):
    return pl.pallas_call(
        body,
        out_shape=[jax.ShapeDtypeStruct((N, D), jnp.float32)] * n_out,
    )


def kernel(features, edge_index, W1, b1, W2, b2, W3, b3, gamma, beta,
           mW1, mb1, mW2, mb2):
    del W2, b2
    src = edge_index[0]
    dst = edge_index[1]
    pad = EP - E
    srcp = jnp.concatenate([src, jnp.zeros((pad,), jnp.int32)])
    dstp = jnp.concatenate([dst, jnp.full((pad,), DUMMY, jnp.int32)])

    zeros = jnp.zeros((ROWS_PT, D), jnp.float32)
    ones = jnp.ones((CHUNK, D), jnp.float32)

    b1r = b1.reshape(1, D)
    b3r = b3.reshape(1, D)
    gammar = gamma.reshape(1, D)
    betar = beta.reshape(1, D)
    mb1r = mb1.reshape(1, D)
    mb2r = mb2.reshape(1, D)

    deg2 = _deg_kernel(dstp, zeros, ones)
    dega = deg2[0, :N, :16]
    degb = deg2[1, :N, :16]

    norm, g1 = _tc_call(_tc1_body, 2)(dega, degb, features)

    a1 = _spmm_kernel(g1, srcp, dstp, zeros)
    tx1, g2 = _tc_call(_tc2_body, 2)(a1[0, :N], a1[1, :N], norm)

    a2 = _spmm_kernel(g2, srcp, dstp, zeros)
    hbn, g3 = _tc_call(_tc3_body, 2)(
        features, tx1, a2[0, :N], a2[1, :N], norm, W1, b1r, gammar, betar)

    a3 = _spmm_kernel(g3, srcp, dstp, zeros)
    t1, g4 = _tc_call(_tc4_body, 2)(a3[0, :N], a3[1, :N], norm)

    a4 = _spmm_kernel(g4, srcp, dstp, zeros)
    (out,) = _tc_call(_tc5_body, 1)(
        hbn, t1, a4[0, :N], a4[1, :N], norm, W3, b3r, mW1, mb1r, mW2, mb2r)
    return out
